# Initial kernel scaffold; baseline (speedup 1.0000x reference)
import jax
import jax.numpy as jnp
from jax.experimental import pallas as pl


def kernel(x, lifecycle, blueprint, strategy, weights):
    raise NotImplementedError("write your pallas kernel here")


# speedup vs baseline: 1.6132x; 1.6132x over previous
import jax
import jax.numpy as jnp
from jax.experimental import pallas as pl
from jax.experimental.pallas import tpu as pltpu

HIDDEN_DIM = 4096
NUM_SEEDS = 16
CHUNK_SIZE = 256
NUM_BLUEPRINTS = 10
ROW_TILE = 512


def _kernel(lc_ref, bp_ref, st_ref, x_ref, w_ref, out_ref, ab_ref):
    @pl.when(pl.program_id(0) == 0)
    def _compute_ab():
        col = jax.lax.broadcasted_iota(jnp.int32, (1, HIDDEN_DIM), 1)
        sid = col // CHUNK_SIZE
        lc = jnp.zeros((1, HIDDEN_DIM), jnp.int32)
        bp = jnp.zeros((1, HIDDEN_DIM), jnp.int32)
        st = jnp.zeros((1, HIDDEN_DIM), jnp.int32)
        for s in range(NUM_SEEDS):
            m = sid == s
            lc = jnp.where(m, lc_ref[s], lc)
            bp = jnp.where(m, bp_ref[s], bp)
            st = jnp.where(m, st_ref[s], st)
        w = jnp.zeros((1, HIDDEN_DIM), jnp.float32)
        for b in range(NUM_BLUEPRINTS):
            w = jnp.where(bp == b, w_ref[b : b + 1, :], w)
        active = (lc >= 3) & (lc <= 6)
        a = jnp.where(st == 0, w, jnp.where(st == 1, 1.0, 0.5))
        bias = jnp.where(st == 0, 0.0, jnp.where(st == 1, w, 0.5 * w))
        ab_ref[0:1, :] = jnp.where(active, a, 1.0)
        ab_ref[1:2, :] = jnp.where(active, bias, 0.0)

    out_ref[...] = x_ref[...] * ab_ref[0:1, :] + ab_ref[1:2, :]


def kernel(x, lifecycle, blueprint, strategy, weights):
    batch, hidden = x.shape
    grid = (batch // ROW_TILE,)
    return pl.pallas_call(
        _kernel,
        grid_spec=pltpu.PrefetchScalarGridSpec(
            num_scalar_prefetch=3,
            grid=grid,
            in_specs=[
                pl.BlockSpec((ROW_TILE, hidden), lambda i, *_: (i, 0)),
                pl.BlockSpec((NUM_BLUEPRINTS, hidden), lambda i, *_: (0, 0)),
            ],
            out_specs=pl.BlockSpec((ROW_TILE, hidden), lambda i, *_: (i, 0)),
            scratch_shapes=[pltpu.VMEM((2, hidden), jnp.float32)],
        ),
        out_shape=jax.ShapeDtypeStruct((batch, hidden), x.dtype),
        compiler_params=pltpu.CompilerParams(
            dimension_semantics=("arbitrary",),
        ),
    )(lifecycle, blueprint, strategy, x, weights)
